# baseline (speedup 1.0000x reference)
import functools
import jax
import jax.numpy as jnp
from jax.experimental import pallas as pl
from jax.experimental.pallas import tpu as pltpu

_NUM_EXPERTS = 8
_BLOCK_ROWS = 8192
_OUT_ROWS = 4096


def _router_block(hs_ref, wt_ref, bias_ref, scores_ref, idx_ref):
    j = pl.program_id(1)
    x = hs_ref[pl.ds(j * _OUT_ROWS, _OUT_ROWS), :]
    wt = wt_ref[...]
    logits = jax.lax.dot_general(
        x, wt, (((1,), (0,)), ((), ())),
        preferred_element_type=jnp.float32,
    )
    logits = logits + bias_ref[...]
    r, e = logits.shape

    lane = jax.lax.broadcasted_iota(jnp.int32, (r, e), 1)
    bits = jax.lax.bitcast_convert_type(logits, jnp.int32)
    key = jax.lax.bitcast_convert_type((bits & -8) | (7 - lane), jnp.float32)

    m1 = jnp.max(key, axis=1, keepdims=True)
    is1 = key == m1
    m2 = jnp.max(jnp.where(is1, -jnp.inf, key), axis=1, keepdims=True)
    is2 = key == m2

    m1b = jax.lax.bitcast_convert_type(m1, jnp.int32)
    m2b = jax.lax.bitcast_convert_type(m2, jnp.int32)
    v1 = jax.lax.bitcast_convert_type(m1b & -8, jnp.float32)
    v2 = jax.lax.bitcast_convert_type(m2b & -8, jnp.float32)

    z = jnp.exp(v2 - v1)
    s1 = 1.0 / (1.0 + z)
    s2 = z * s1

    scores_ref[...] = jnp.where(is1, s1, jnp.where(is2, s2, 0.0))
    idx_ref[...] = jnp.concatenate(
        [7 - (m1b & 7), 7 - (m2b & 7)], axis=1)


@jax.jit
def kernel(hidden_states, weight, bias):
    hidden = weight.shape[1]
    hs = hidden_states.reshape(-1, hidden)
    n = hs.shape[0]
    e = weight.shape[0]
    grid = (n // _BLOCK_ROWS, _BLOCK_ROWS // _OUT_ROWS)

    scores, indices = pl.pallas_call(
        _router_block,
        grid=grid,
        in_specs=[
            pl.BlockSpec((_BLOCK_ROWS, hidden), lambda i, j: (i, 0)),
            pl.BlockSpec((hidden, e), lambda i, j: (0, 0)),
            pl.BlockSpec((1, e), lambda i, j: (0, 0)),
        ],
        out_specs=[
            pl.BlockSpec((_OUT_ROWS, e), lambda i, j: (2 * i + j, 0)),
            pl.BlockSpec((_OUT_ROWS, 2), lambda i, j: (2 * i + j, 0)),
        ],
        out_shape=[
            jax.ShapeDtypeStruct((n, e), jnp.float32),
            jax.ShapeDtypeStruct((n, 2), jnp.int32),
        ],
        compiler_params=pltpu.CompilerParams(
            dimension_semantics=("arbitrary", "arbitrary"),
        ),
    )(hs, weight.T, bias.reshape(1, e))
    return scores, indices


# speedup vs baseline: 1.2021x; 1.2021x over previous
import functools
import jax
import jax.numpy as jnp
from jax import lax
from jax.experimental import pallas as pl
from jax.experimental.pallas import tpu as pltpu

_NUM_EXPERTS = 8
_CH = 2048
_N = 32768
_NCH = _N // _CH
_NBUF = 4
_NGRP = _NCH // _NBUF


def _topk_epilogue(logits):
    r, e = logits.shape
    lane = jax.lax.broadcasted_iota(jnp.int32, (r, e), 1)
    bits = jax.lax.bitcast_convert_type(logits, jnp.int32)
    key = jax.lax.bitcast_convert_type((bits & -8) | (7 - lane), jnp.float32)
    m1 = jnp.max(key, axis=1, keepdims=True)
    is1 = key == m1
    m2 = jnp.max(jnp.where(is1, -jnp.inf, key), axis=1, keepdims=True)
    is2 = key == m2
    m1b = jax.lax.bitcast_convert_type(m1, jnp.int32)
    m2b = jax.lax.bitcast_convert_type(m2, jnp.int32)
    v1 = jax.lax.bitcast_convert_type(m1b & -8, jnp.float32)
    v2 = jax.lax.bitcast_convert_type(m2b & -8, jnp.float32)
    z = jnp.exp(v2 - v1)
    s1 = 1.0 / (1.0 + z)
    s2 = z * s1
    scores = jnp.where(is1, s1, jnp.where(is2, s2, 0.0))
    idx = jnp.concatenate([7 - (m1b & 7), 7 - (m2b & 7)], axis=1)
    return scores, idx


def _body(hs, wt_ref, bias_ref, scores, idx,
          xbuf, sbuf, ibuf, insem, ssem, isem):
    wt = wt_ref[...]
    bias = bias_ref[...]

    def in_cp(c, b):
        return pltpu.make_async_copy(
            hs.at[pl.ds(c * _CH, _CH), :], xbuf.at[b], insem.at[b])

    def s_cp(c, b):
        return pltpu.make_async_copy(
            sbuf.at[b], scores.at[pl.ds(c * _CH, _CH), :], ssem.at[b])

    def i_cp(c, b):
        return pltpu.make_async_copy(
            ibuf.at[b], idx.at[pl.ds(c * _CH, _CH), :], isem.at[b])

    for b in range(_NBUF):
        in_cp(b, b).start()

    def grp(g, carry):
        for b in range(_NBUF):
            c = g * _NBUF + b
            in_cp(c, b).wait()

            @pl.when(g > 0)
            def _():
                s_cp(c, b).wait()
                i_cp(c, b).wait()

            logits = jax.lax.dot_general(
                xbuf[b], wt, (((1,), (0,)), ((), ())),
                preferred_element_type=jnp.float32,
            ) + bias
            s, ix = _topk_epilogue(logits)
            sbuf[b] = s
            ibuf[b] = ix
            s_cp(c, b).start()
            i_cp(c, b).start()

            @pl.when(g < _NGRP - 1)
            def _():
                in_cp(c + _NBUF, b).start()
        return carry

    lax.fori_loop(0, _NGRP, grp, 0)
    for b in range(_NBUF):
        s_cp(_NCH - _NBUF + b, b).wait()
        i_cp(_NCH - _NBUF + b, b).wait()


@jax.jit
def kernel(hidden_states, weight, bias):
    hidden = weight.shape[1]
    hs = hidden_states.reshape(-1, hidden)
    n = hs.shape[0]
    e = weight.shape[0]

    scores, indices = pl.pallas_call(
        _body,
        in_specs=[
            pl.BlockSpec(memory_space=pltpu.MemorySpace.HBM),
            pl.BlockSpec(memory_space=pltpu.MemorySpace.VMEM),
            pl.BlockSpec(memory_space=pltpu.MemorySpace.VMEM),
        ],
        out_specs=[
            pl.BlockSpec(memory_space=pltpu.MemorySpace.HBM),
            pl.BlockSpec(memory_space=pltpu.MemorySpace.HBM),
        ],
        out_shape=[
            jax.ShapeDtypeStruct((n, e), jnp.float32),
            jax.ShapeDtypeStruct((n, 2), jnp.int32),
        ],
        scratch_shapes=[
            pltpu.VMEM((_NBUF, _CH, hidden), jnp.float32),
            pltpu.VMEM((_NBUF, _CH, _NUM_EXPERTS), jnp.float32),
            pltpu.VMEM((_NBUF, _CH, 2), jnp.int32),
            pltpu.SemaphoreType.DMA((_NBUF,)),
            pltpu.SemaphoreType.DMA((_NBUF,)),
            pltpu.SemaphoreType.DMA((_NBUF,)),
        ],
    )(hs, weight.T, bias.reshape(1, e))
    return scores, indices


# speedup vs baseline: 1.2028x; 1.0006x over previous
import functools
import jax
import jax.numpy as jnp
from jax import lax
from jax.experimental import pallas as pl
from jax.experimental.pallas import tpu as pltpu

_NUM_EXPERTS = 8
_CH = 2048
_N = 32768
_NCH = _N // _CH
_NBUF = 4
_NGRP = _NCH // _NBUF


def _topk_epilogue(logits):
    r, e = logits.shape
    lane = jax.lax.broadcasted_iota(jnp.int32, (r, e), 1)
    bits = jax.lax.bitcast_convert_type(logits, jnp.int32)
    key = jax.lax.bitcast_convert_type((bits & -8) | (7 - lane), jnp.float32)
    m1 = jnp.max(key, axis=1, keepdims=True)
    is1 = key == m1
    m2 = jnp.max(jnp.where(is1, -jnp.inf, key), axis=1, keepdims=True)
    is2 = key == m2
    m1b = jax.lax.bitcast_convert_type(m1, jnp.int32)
    m2b = jax.lax.bitcast_convert_type(m2, jnp.int32)
    v1 = jax.lax.bitcast_convert_type(m1b & -8, jnp.float32)
    v2 = jax.lax.bitcast_convert_type(m2b & -8, jnp.float32)
    z = jnp.exp(v2 - v1)
    s1 = 1.0 / (1.0 + z)
    s2 = z * s1
    scores = jnp.where(is1, s1, jnp.where(is2, s2, 0.0))
    idx = jnp.concatenate([7 - (m1b & 7), 7 - (m2b & 7)], axis=1)
    return scores, idx


def _body(hs, wt_ref, bias_ref, scores, idx,
          xbuf, sbuf, ibuf, insem, insem2, ssem, isem):
    wt = wt_ref[...]
    bias = bias_ref[...]

    half = _CH // 2

    def in_cp_a(c, b):
        return pltpu.make_async_copy(
            hs.at[pl.ds(c * _CH, half), :],
            xbuf.at[b, pl.ds(0, half)], insem.at[b])

    def in_cp_b(c, b):
        return pltpu.make_async_copy(
            hs.at[pl.ds(c * _CH + half, half), :],
            xbuf.at[b, pl.ds(half, half)], insem2.at[b])

    def in_start(c, b):
        in_cp_a(c, b).start()
        in_cp_b(c, b).start()

    def in_wait(c, b):
        in_cp_a(c, b).wait()
        in_cp_b(c, b).wait()

    def s_cp(c, b):
        return pltpu.make_async_copy(
            sbuf.at[b], scores.at[pl.ds(c * _CH, _CH), :], ssem.at[b])

    def i_cp(c, b):
        return pltpu.make_async_copy(
            ibuf.at[b], idx.at[pl.ds(c * _CH, _CH), :], isem.at[b])

    for b in range(_NBUF):
        in_start(b, b)

    def grp(g, carry):
        for b in range(_NBUF):
            c = g * _NBUF + b
            in_wait(c, b)

            @pl.when(g > 0)
            def _():
                s_cp(c, b).wait()
                i_cp(c, b).wait()

            logits = jax.lax.dot_general(
                xbuf[b], wt, (((1,), (0,)), ((), ())),
                preferred_element_type=jnp.float32,
            ) + bias
            s, ix = _topk_epilogue(logits)
            sbuf[b] = s
            ibuf[b] = ix
            s_cp(c, b).start()
            i_cp(c, b).start()

            @pl.when(g < _NGRP - 1)
            def _():
                in_start(c + _NBUF, b)
        return carry

    lax.fori_loop(0, _NGRP, grp, 0)
    for b in range(_NBUF):
        s_cp(_NCH - _NBUF + b, b).wait()
        i_cp(_NCH - _NBUF + b, b).wait()


@jax.jit
def kernel(hidden_states, weight, bias):
    hidden = weight.shape[1]
    hs = hidden_states.reshape(-1, hidden)
    n = hs.shape[0]
    e = weight.shape[0]

    scores, indices = pl.pallas_call(
        _body,
        in_specs=[
            pl.BlockSpec(memory_space=pltpu.MemorySpace.HBM),
            pl.BlockSpec(memory_space=pltpu.MemorySpace.VMEM),
            pl.BlockSpec(memory_space=pltpu.MemorySpace.VMEM),
        ],
        out_specs=[
            pl.BlockSpec(memory_space=pltpu.MemorySpace.HBM),
            pl.BlockSpec(memory_space=pltpu.MemorySpace.HBM),
        ],
        out_shape=[
            jax.ShapeDtypeStruct((n, e), jnp.float32),
            jax.ShapeDtypeStruct((n, 2), jnp.int32),
        ],
        scratch_shapes=[
            pltpu.VMEM((_NBUF, _CH, hidden), jnp.float32),
            pltpu.VMEM((_NBUF, _CH, _NUM_EXPERTS), jnp.float32),
            pltpu.VMEM((_NBUF, _CH, 2), jnp.int32),
            pltpu.SemaphoreType.DMA((_NBUF,)),
            pltpu.SemaphoreType.DMA((_NBUF,)),
            pltpu.SemaphoreType.DMA((_NBUF,)),
            pltpu.SemaphoreType.DMA((_NBUF,)),
        ],
    )(hs, weight.T, bias.reshape(1, e))
    return scores, indices
